# Initial kernel scaffold; baseline (speedup 1.0000x reference)
import jax
import jax.numpy as jnp
from jax.experimental import pallas as pl


def kernel(x, w1, b1, wh, bh, wt, bt, wl1, bl1, wl2, bl2, wa1, ba1, wa2, ba2, wfc, bfc, ln_w, ln_b):
    raise NotImplementedError("write your pallas kernel here")


# speedup vs baseline: 10.1439x; 10.1439x over previous
import functools

import jax
import jax.numpy as jnp
from jax import lax
from jax.experimental import pallas as pl
from jax.experimental.pallas import tpu as pltpu
from jax.experimental.pallas import tpu_sc as plsc

N = 4096
D0 = 1024
D1 = 512
TOPK = 6
BLK = 256
NBLK = N // BLK
SCALE = D1 ** (-0.5)
NEG_SLOPE = 0.01


def _leaky_relu(v):
    return jnp.where(v >= 0, v, NEG_SLOPE * v)


def _fc1_body(x_ref, w_ref, b_ref, x1_ref, cs_ref):
    i = pl.program_id(0)
    x1 = jnp.maximum(
        jnp.dot(x_ref[...], w_ref[...], preferred_element_type=jnp.float32)
        + b_ref[...], 0.0)
    x1_ref[...] = x1

    @pl.when(i == 0)
    def _():
        cs_ref[...] = jnp.zeros_like(cs_ref)

    cs_ref[...] += jnp.sum(x1, axis=0, keepdims=True)


def _fc1(x, w1t, b1):
    return pl.pallas_call(
        _fc1_body,
        grid=(NBLK,),
        in_specs=[
            pl.BlockSpec((BLK, D0), lambda i: (i, 0)),
            pl.BlockSpec((D0, D1), lambda i: (0, 0)),
            pl.BlockSpec((1, D1), lambda i: (0, 0)),
        ],
        out_specs=[
            pl.BlockSpec((BLK, D1), lambda i: (i, 0)),
            pl.BlockSpec((1, D1), lambda i: (0, 0)),
        ],
        out_shape=[
            jax.ShapeDtypeStruct((N, D1), jnp.float32),
            jax.ShapeDtypeStruct((1, D1), jnp.float32),
        ],
    )(x, w1t, b1)


def _heads_body(x1_ref, cs_ref, wht_ref, bh_ref, wtt_ref, bt_ref,
                eh_ref, et_ref):
    mean = cs_ref[...] * (1.0 / N)
    x2 = (x1_ref[...] + mean) * 0.5
    eh_ref[...] = jnp.dot(x2, wht_ref[...],
                          preferred_element_type=jnp.float32) + bh_ref[...]
    et_ref[...] = jnp.dot(x2, wtt_ref[...],
                          preferred_element_type=jnp.float32) + bt_ref[...]


def _heads(x1, cs, wht, bh, wtt, bt):
    return pl.pallas_call(
        _heads_body,
        grid=(NBLK,),
        in_specs=[
            pl.BlockSpec((BLK, D1), lambda i: (i, 0)),
            pl.BlockSpec((1, D1), lambda i: (0, 0)),
            pl.BlockSpec((D1, D1), lambda i: (0, 0)),
            pl.BlockSpec((1, D1), lambda i: (0, 0)),
            pl.BlockSpec((D1, D1), lambda i: (0, 0)),
            pl.BlockSpec((1, D1), lambda i: (0, 0)),
        ],
        out_specs=[
            pl.BlockSpec((BLK, D1), lambda i: (i, 0)),
            pl.BlockSpec((BLK, D1), lambda i: (i, 0)),
        ],
        out_shape=[
            jax.ShapeDtypeStruct((N, D1), jnp.float32),
            jax.ShapeDtypeStruct((N, D1), jnp.float32),
        ],
    )(x1, cs, wht, bh, wtt, bt)


def _topk_body(eh_ref, et_ref, idx_ref):
    ehs = eh_ref[...] * SCALE
    logits = lax.dot_general(
        ehs, et_ref[...],
        dimension_numbers=(((1,), (1,)), ((), ())),
        preferred_element_type=jnp.float32)
    iota = lax.broadcasted_iota(jnp.int32, (BLK, N), 1)
    vals = logits
    cols = []
    for _ in range(TOPK):
        m = jnp.max(vals, axis=1, keepdims=True)
        cand = jnp.where(vals == m, iota, N)
        idx = jnp.min(cand, axis=1, keepdims=True)
        cols.append(idx)
        vals = jnp.where(iota == idx, -jnp.inf, vals)
    idx_ref[...] = jnp.concatenate(cols, axis=1)


def _topk(e_h, e_t):
    return pl.pallas_call(
        _topk_body,
        grid=(NBLK,),
        in_specs=[
            pl.BlockSpec((BLK, D1), lambda i: (i, 0)),
            pl.BlockSpec((N, D1), lambda i: (0, 0)),
        ],
        out_specs=pl.BlockSpec((BLK, TOPK), lambda i: (i, 0)),
        out_shape=jax.ShapeDtypeStruct((N, TOPK), jnp.int32),
    )(e_h, e_t)


_SC_NC = 2
_SC_NS = 16
_SC_NW = _SC_NC * _SC_NS
_SC_B = TOPK * N
_SC_BPW = _SC_B // _SC_NW
_SC_R = 96


def _sc_gather_body(table_hbm, idx_hbm, out_hbm, idx_v, rows_v, sem):
    wid = lax.axis_index("s") * _SC_NC + lax.axis_index("c")
    base = wid * _SC_BPW
    pltpu.sync_copy(idx_hbm.at[pl.ds(base, _SC_BPW)], idx_v)
    for c in range(_SC_BPW // _SC_R):
        pltpu.async_copy(
            table_hbm.at[idx_v.at[pl.ds(c * _SC_R, _SC_R)]], rows_v, sem
        ).wait()
        pltpu.sync_copy(rows_v, out_hbm.at[pl.ds(base + c * _SC_R, _SC_R)])


def _sc_gather(table, idx_flat):
    mesh = plsc.VectorSubcoreMesh(core_axis_name="c", subcore_axis_name="s")
    fn = functools.partial(
        pl.kernel,
        out_type=jax.ShapeDtypeStruct((_SC_B, D1), jnp.float32),
        mesh=mesh,
        scratch_types=[
            pltpu.VMEM((_SC_BPW,), jnp.int32),
            pltpu.VMEM((_SC_R, D1), jnp.float32),
            pltpu.SemaphoreType.DMA,
        ],
    )(_sc_gather_body)
    return fn(table, idx_flat)


def _gate_body(eh_ref, nb_ref, wl1t_ref, bl1_ref, wl2t_ref, bl2_ref,
               wa1t_ref, ba1_ref, wa2t_ref, ba2_ref, h_ref, att_ref):
    eh = eh_ref[...]
    nbs = [nb_ref[k] for k in range(TOPK)]
    tw = jnp.concatenate(
        [SCALE * jnp.sum(eh * nb, axis=1, keepdims=True) for nb in nbs],
        axis=1)
    tw_m = jnp.max(tw, axis=1, keepdims=True)
    tw_e = jnp.exp(tw - tw_m)
    tp = tw_e / jnp.sum(tw_e, axis=1, keepdims=True)
    kas = []
    for k in range(TOPK):
        p = tp[:, k:k + 1]
        gate = jnp.tanh((2.0 - p) * eh + p * nbs[k])
        kas.append(jnp.sum(nbs[k], axis=1, keepdims=True)
                   * jnp.sum(gate, axis=1, keepdims=True))
    ka = jnp.concatenate(kas, axis=1)
    ka_m = jnp.max(ka, axis=1, keepdims=True)
    ka_e = jnp.exp(ka - ka_m)
    kp = ka_e / jnp.sum(ka_e, axis=1, keepdims=True)
    e_nh = kp[:, 0:1] * nbs[0]
    for k in range(1, TOPK):
        e_nh += kp[:, k:k + 1] * nbs[k]
    se = _leaky_relu(jnp.dot(eh + e_nh, wl1t_ref[...],
                             preferred_element_type=jnp.float32) + bl1_ref[...])
    be = _leaky_relu(jnp.dot(eh * e_nh, wl2t_ref[...],
                             preferred_element_type=jnp.float32) + bl2_ref[...])
    h = se + be
    h_ref[...] = h
    a1 = _leaky_relu(jnp.dot(h, wa1t_ref[...],
                             preferred_element_type=jnp.float32) + ba1_ref[...])
    att_ref[...] = jnp.dot(a1, wa2t_ref[...],
                           preferred_element_type=jnp.float32) + ba2_ref[...]


def _gate(e_h, nb, wl1t, bl1, wl2t, bl2, wa1t, ba1, wa2t, ba2):
    return pl.pallas_call(
        _gate_body,
        grid=(NBLK,),
        in_specs=[
            pl.BlockSpec((BLK, D1), lambda i: (i, 0)),
            pl.BlockSpec((TOPK, BLK, D1), lambda i: (0, i, 0)),
            pl.BlockSpec((D1, D1), lambda i: (0, 0)),
            pl.BlockSpec((1, D1), lambda i: (0, 0)),
            pl.BlockSpec((D1, D1), lambda i: (0, 0)),
            pl.BlockSpec((1, D1), lambda i: (0, 0)),
            pl.BlockSpec((D1, D1 // 2), lambda i: (0, 0)),
            pl.BlockSpec((1, D1 // 2), lambda i: (0, 0)),
            pl.BlockSpec((D1 // 2, 1), lambda i: (0, 0)),
            pl.BlockSpec((1, 1), lambda i: (0, 0)),
        ],
        out_specs=[
            pl.BlockSpec((BLK, D1), lambda i: (i, 0)),
            pl.BlockSpec((BLK, 1), lambda i: (i, 0)),
        ],
        out_shape=[
            jax.ShapeDtypeStruct((N, D1), jnp.float32),
            jax.ShapeDtypeStruct((N, 1), jnp.float32),
        ],
    )(e_h, nb, wl1t, bl1, wl2t, bl2, wa1t, ba1, wa2t, ba2)


def _readout_body(h_ref, att_ref, lnw_ref, lnb_ref, wfct_ref, bfc_ref,
                  out_ref, prob_ref, yhat_ref):
    att = att_ref[...]
    am = jnp.max(att, axis=0, keepdims=True)
    ae = jnp.exp(att - am)
    aw = ae / jnp.sum(ae, axis=0, keepdims=True)
    ws = jnp.sum(aw * h_ref[...], axis=0, keepdims=True)
    mu = jnp.mean(ws, axis=1, keepdims=True)
    var = jnp.mean((ws - mu) ** 2, axis=1, keepdims=True)
    hn = (ws - mu) / jnp.sqrt(var + 1e-5) * lnw_ref[...] + lnb_ref[...]
    logit = jnp.dot(hn, wfct_ref[...],
                    preferred_element_type=jnp.float32) + bfc_ref[...]
    out_ref[...] = logit
    lm = jnp.max(logit, axis=1, keepdims=True)
    le = jnp.exp(logit - lm)
    prob_ref[...] = le / jnp.sum(le, axis=1, keepdims=True)
    yhat_ref[...] = jnp.where(logit[0:1, 1:2] > logit[0:1, 0:1], 1, 0
                              ).astype(jnp.int32)


def _readout(h, att, ln_w2, ln_b2, wfct, bfc):
    return pl.pallas_call(
        _readout_body,
        grid=(1,),
        in_specs=[
            pl.BlockSpec((N, D1), lambda i: (0, 0)),
            pl.BlockSpec((N, 1), lambda i: (0, 0)),
            pl.BlockSpec((1, D1), lambda i: (0, 0)),
            pl.BlockSpec((1, D1), lambda i: (0, 0)),
            pl.BlockSpec((D1, 2), lambda i: (0, 0)),
            pl.BlockSpec((1, 2), lambda i: (0, 0)),
        ],
        out_specs=[
            pl.BlockSpec((1, 2), lambda i: (0, 0)),
            pl.BlockSpec((1, 2), lambda i: (0, 0)),
            pl.BlockSpec((1, 1), lambda i: (0, 0)),
        ],
        out_shape=[
            jax.ShapeDtypeStruct((1, 2), jnp.float32),
            jax.ShapeDtypeStruct((1, 2), jnp.float32),
            jax.ShapeDtypeStruct((1, 1), jnp.int32),
        ],
    )(h, att, ln_w2, ln_b2, wfct, bfc)


def kernel(x, w1, b1, wh, bh, wt, bt, wl1, bl1, wl2, bl2,
           wa1, ba1, wa2, ba2, wfc, bfc, ln_w, ln_b):
    w1t = w1.T
    wht, wtt = wh.T, wt.T
    wl1t, wl2t = wl1.T, wl2.T
    wa1t, wa2t = wa1.T, wa2.T
    wfct = wfc.T
    b1r = b1[None]
    bhr, btr = bh[None], bt[None]
    bl1r, bl2r = bl1[None], bl2[None]
    ba1r, ba2r = ba1[None], ba2[None]
    bfcr = bfc[None]
    lnwr, lnbr = ln_w[None], ln_b[None]

    x1, cs = _fc1(x, w1t, b1r)
    e_h, e_t = _heads(x1, cs, wht, bhr, wtt, btr)
    idx = _topk(e_h, e_t)
    idx_flat = idx.T.reshape(-1)
    nb_flat = _sc_gather(e_t, idx_flat)
    nb = nb_flat.reshape(TOPK, N, D1)
    h, att = _gate(e_h, nb, wl1t, bl1r, wl2t, bl2r, wa1t, ba1r, wa2t, ba2r)
    out, prob, yhat = _readout(h, att, lnwr, lnbr, wfct, bfcr)
    return out, prob, yhat
